# baseline (speedup 1.0000x reference)
import functools

import jax
import jax.numpy as jnp
from jax.experimental import pallas as pl
from jax.experimental.pallas import tpu as pltpu

H = 1024
E = 8
TOPK = 2
EPS = 1e-06
NT = 2


def _moe_body(xb_ref, wg_ref, bg_ref, we_ref, be_ref,
              out_ref, aux_ref, wd_ref, y_ref):
    t = pl.program_id(0)
    e = pl.program_id(1)
    n = xb_ref.shape[0]
    hn = n // NT
    row0 = t * hn

    @pl.when((t == 0) & (e == 0))
    def _router():
        logits = jax.lax.dot_general(
            xb_ref[...], wg_ref[...], (((1,), (1,)), ((), ())),
            precision=jax.lax.Precision.DEFAULT,
            preferred_element_type=jnp.float32) + bg_ref[...][None, :]
        m = jnp.max(logits, axis=1, keepdims=True)
        ex = jnp.exp(logits - m)
        probs = ex / jnp.sum(ex, axis=1, keepdims=True)
        iota = jax.lax.broadcasted_iota(jnp.int32, (n, E), 1)
        p1 = jnp.max(probs, axis=1, keepdims=True)
        i1 = jnp.min(jnp.where(probs == p1, iota, E), axis=1, keepdims=True)
        masked = jnp.where(iota == i1, -jnp.inf, probs)
        p2 = jnp.max(masked, axis=1, keepdims=True)
        i2 = jnp.min(jnp.where(masked == p2, iota, E), axis=1, keepdims=True)
        denom = p1 + p2 + EPS
        w1 = p1 / denom
        w2 = p2 / denom
        wd_ref[...] = (jnp.where(iota == i1, w1, 0.0)
                       + jnp.where(iota == i2, w2, 0.0))
        mask = ((iota == i1) | (iota == i2)).astype(jnp.float32)
        usage = jnp.mean(mask, axis=0)
        gates = jnp.mean(probs, axis=0)
        aux_ref[0, 0] = jnp.sum(usage * gates) * E

    y_ref[e] = jax.lax.dot_general(
        xb_ref[pl.ds(row0, hn), :], we_ref[0].astype(jnp.bfloat16),
        (((1,), (1,)), ((), ())),
        preferred_element_type=jnp.float32).astype(jnp.bfloat16)

    @pl.when(e == E - 1)
    def _combine():
        wdh = wd_ref[pl.ds(row0, hn), :]
        acc = jax.lax.dot_general(
            wdh, be_ref[...], (((1,), (0,)), ((), ())),
            precision=jax.lax.Precision.DEFAULT,
            preferred_element_type=jnp.float32)
        for ee in range(E):
            acc += wdh[:, ee:ee + 1] * y_ref[ee].astype(jnp.float32)
        out_ref[pl.ds(row0, hn), :] = acc


@jax.jit
def kernel(x, Wg, bg, We, be):
    b, s, h = x.shape
    xb = x.reshape(-1, h).astype(jnp.bfloat16)
    n = xb.shape[0]

    out, aux = pl.pallas_call(
        _moe_body,
        grid=(NT, E),
        in_specs=[
            pl.BlockSpec((n, h), lambda t, e: (0, 0)),
            pl.BlockSpec((E, h), lambda t, e: (0, 0)),
            pl.BlockSpec((E,), lambda t, e: (0,)),
            pl.BlockSpec((1, h, h), lambda t, e: (e, 0, 0)),
            pl.BlockSpec((E, h), lambda t, e: (0, 0)),
        ],
        out_specs=[
            pl.BlockSpec((n, h), lambda t, e: (0, 0)),
            pl.BlockSpec(memory_space=pltpu.SMEM),
        ],
        out_shape=[
            jax.ShapeDtypeStruct((n, h), jnp.float32),
            jax.ShapeDtypeStruct((1, 1), jnp.float32),
        ],
        scratch_shapes=[
            pltpu.VMEM((n, E), jnp.float32),
            pltpu.VMEM((E, n // NT, h), jnp.bfloat16),
        ],
    )(xb, Wg, bg, We, be)

    return out.reshape(b, s, h), aux[0, 0]
